# Initial kernel scaffold; baseline (speedup 1.0000x reference)
import jax
import jax.numpy as jnp
from jax.experimental import pallas as pl


def kernel(x_flow, port_table, tcp_table, Wl1_hf, Wr1_hf, b1_hf, Wl1_fh, Wr1_fh, b1_fh, Wl1_ff, Wr1_ff, b1_ff, Wl2_hf, Wr2_hf, b2_hf, Wl2_ff, Wr2_ff, b2_ff, Wc1, bc1, Wc2, bc2, Wc3, bc3, dst_ports, tcp_flags, edge_index_hf, edge_index_fh, edge_index_ff, batch, num_hosts):
    raise NotImplementedError("write your pallas kernel here")


# speedup vs baseline: 6.7758x; 6.7758x over previous
import functools

import jax
import jax.numpy as jnp
from jax import lax
from jax.experimental import pallas as pl
from jax.experimental.pallas import tpu as pltpu
from jax.experimental.pallas import tpu_sc as plsc

NF = 50000
NH = 10000
NG = 64
H = 64

CH = 125
SPC = 8
CHUNK = CH * SPC
HCHUNK = CHUNK // 2


_MESH = plsc.VectorSubcoreMesh(core_axis_name="c", subcore_axis_name="s")
_NSUB = 16
_NWORK = 32


def _ceil_div(a, b):
    return (a + b - 1) // b


def _sc_embed_counts(ports2, tcps2, eff3, efh3, ehf3, ptab, ttab,
                     ones_hbm, zeros_hbm):
    n_pe = NF // CHUNK
    n_ff = (12 * NF) // CHUNK
    n_h = NH * 10 // CHUNK
    n_zf = NF // CHUNK
    n_zh = NH // CHUNK

    def body(ports2, tcps2, eff3, efh3, ehf3, ptab, ttab, ones_hbm,
             zeros_hbm, pe_out, te_out, cnt_ff, cnt_fh, cnt_hf,
             idx2, rows_pe, rows_te, ones_v, zrows, acc_ff, acc_fh, acc_hf,
             sem):
        c = lax.axis_index("c")
        s = lax.axis_index("s")
        w = s * 2 + c

        pltpu.sync_copy(ones_hbm, ones_v)
        pltpu.sync_copy(zeros_hbm, zrows)

        @pl.when(c == 0)
        def _():
            for k in range(_ceil_div(n_zf, _NSUB)):
                ch = s + _NSUB * k
                @pl.when(ch < n_zf)
                def _():
                    pltpu.sync_copy(zrows, acc_ff.at[pl.ds(ch * CHUNK, CHUNK), :])

        @pl.when(c == 1)
        def _():
            @pl.when(s < n_zh)
            def _():
                pltpu.sync_copy(zrows, acc_fh.at[pl.ds(s * CHUNK, CHUNK), :])
                pltpu.sync_copy(zrows, acc_hf.at[pl.ds(s * CHUNK, CHUNK), :])

        plsc.subcore_barrier()

        for k in range(_ceil_div(n_pe, _NWORK)):
            ch = w + _NWORK * k
            @pl.when(ch < n_pe)
            def _():
                pltpu.sync_copy(ports2.at[pl.ds(ch * SPC, SPC), :], idx2)
                for j in range(SPC):
                    pltpu.async_copy(ptab.at[idx2.at[j]],
                                     rows_pe.at[pl.ds(j * CH, CH), :],
                                     sem).wait()
                pltpu.sync_copy(rows_pe, pe_out.at[pl.ds(ch * CHUNK, CHUNK), :])

        for k in range(_ceil_div(n_pe, _NWORK)):
            ch = w + _NWORK * k
            @pl.when(ch < n_pe)
            def _():
                pltpu.sync_copy(tcps2.at[pl.ds(ch * SPC, SPC), :], idx2)
                for j in range(SPC):
                    pltpu.async_copy(ttab.at[idx2.at[j]],
                                     rows_te.at[pl.ds(j * CH, CH), :],
                                     sem).wait()
                pltpu.sync_copy(rows_te, te_out.at[pl.ds(ch * CHUNK, CHUNK), :])

        @pl.when(c == 0)
        def _():
            for k in range(_ceil_div(n_ff, _NSUB)):
                ch = s + _NSUB * k
                @pl.when(ch < n_ff)
                def _():
                    pltpu.sync_copy(eff3.at[pl.ds(4800 + ch * SPC, SPC), :], idx2)
                    for j in range(SPC):
                        pltpu.sync_copy(ones_v, acc_ff.at[idx2.at[j]], add=True)

        @pl.when(c == 1)
        def _():
            for k in range(_ceil_div(n_h, _NSUB)):
                ch = s + _NSUB * k
                @pl.when(ch < n_h)
                def _():
                    pltpu.sync_copy(efh3.at[pl.ds(800 + ch * SPC, SPC), :], idx2)
                    for j in range(SPC):
                        pltpu.sync_copy(ones_v, acc_fh.at[idx2.at[j]], add=True)
            for k in range(_ceil_div(n_h, _NSUB)):
                ch = s + _NSUB * k
                @pl.when(ch < n_h)
                def _():
                    pltpu.sync_copy(ehf3.at[pl.ds(800 + ch * SPC, SPC), :], idx2)
                    for j in range(SPC):
                        pltpu.sync_copy(ones_v, acc_hf.at[idx2.at[j]], add=True)

        plsc.subcore_barrier()

        @pl.when(c == 0)
        def _():
            for k in range(_ceil_div(n_zf, _NSUB)):
                ch = s + _NSUB * k
                @pl.when(ch < n_zf)
                def _():
                    sl = pl.ds(ch * CHUNK, CHUNK)
                    pltpu.sync_copy(acc_ff.at[sl, :], cnt_ff.at[sl, :])

        @pl.when(c == 1)
        def _():
            @pl.when(s < n_zh)
            def _():
                sl = pl.ds(s * CHUNK, CHUNK)
                pltpu.sync_copy(acc_fh.at[sl, :], cnt_fh.at[sl, :])
                pltpu.sync_copy(acc_hf.at[sl, :], cnt_hf.at[sl, :])

    fn = pl.kernel(
        body,
        out_type=[
            jax.ShapeDtypeStruct((NF, 8), jnp.float32),
            jax.ShapeDtypeStruct((NF, 8), jnp.float32),
            jax.ShapeDtypeStruct((NF, 16), jnp.float32),
            jax.ShapeDtypeStruct((NH, 16), jnp.float32),
            jax.ShapeDtypeStruct((NH, 16), jnp.float32),
        ],
        mesh=_MESH,
        compiler_params=pltpu.CompilerParams(use_tc_tiling_on_sc=False),
        scratch_types=[
            pltpu.VMEM((SPC, CH), jnp.int32),
            pltpu.VMEM((CHUNK, 8), jnp.float32),
            pltpu.VMEM((CHUNK, 8), jnp.float32),
            pltpu.VMEM((CH, 16), jnp.float32),
            pltpu.VMEM((CHUNK, 16), jnp.float32),
            pltpu.VMEM_SHARED((NF, 16), jnp.float32),
            pltpu.VMEM_SHARED((NH, 16), jnp.float32),
            pltpu.VMEM_SHARED((NH, 16), jnp.float32),
            pltpu.SemaphoreType.DMA,
        ],
    )
    return fn(ports2, tcps2, eff3, efh3, ehf3, ptab, ttab, ones_hbm,
              zeros_hbm)


def _sc_segsum(e3, y0, y1, zeros32, n_dst, n_edge):
    nch = n_edge // CHUNK
    zch = n_dst // HCHUNK
    doff = n_edge // CH

    def body(e3, y0, y1, zeros32, s0_out, s1_out,
             idxs, idxd, rows, acc, sem):
        c = lax.axis_index("c")
        s = lax.axis_index("s")

        pltpu.sync_copy(zeros32, rows)
        for k in range(_ceil_div(zch, _NSUB)):
            ch = s + _NSUB * k
            @pl.when(ch < zch)
            def _():
                pltpu.sync_copy(rows, acc.at[pl.ds(ch * HCHUNK, HCHUNK), :])

        plsc.subcore_barrier()

        for k in range(_ceil_div(nch, _NSUB)):
            ch = s + _NSUB * k
            @pl.when(ch < nch)
            def _():
                pltpu.sync_copy(e3.at[pl.ds(ch * SPC, SPC), :], idxs)
                pltpu.sync_copy(e3.at[pl.ds(doff + ch * SPC, SPC), :], idxd)

                for half in range(2):
                    @pl.when(c == 0)
                    def _():
                        for j in range(SPC // 2):
                            pltpu.async_copy(
                                y0.at[idxs.at[half * (SPC // 2) + j]],
                                rows.at[pl.ds(j * CH, CH), :],
                                sem).wait()

                    @pl.when(c == 1)
                    def _():
                        for j in range(SPC // 2):
                            pltpu.async_copy(
                                y1.at[idxs.at[half * (SPC // 2) + j]],
                                rows.at[pl.ds(j * CH, CH), :],
                                sem).wait()

                    for j in range(SPC // 2):
                        pltpu.sync_copy(
                            rows.at[pl.ds(j * CH, CH), :],
                            acc.at[idxd.at[half * (SPC // 2) + j]], add=True)

        plsc.subcore_barrier()

        for k in range(_ceil_div(zch, _NSUB)):
            ch = s + _NSUB * k
            @pl.when(ch < zch)
            def _():
                sl = pl.ds(ch * HCHUNK, HCHUNK)

                @pl.when(c == 0)
                def _():
                    pltpu.sync_copy(acc.at[sl, :], s0_out.at[sl, :])

                @pl.when(c == 1)
                def _():
                    pltpu.sync_copy(acc.at[sl, :], s1_out.at[sl, :])

    fn = pl.kernel(
        body,
        out_type=[
            jax.ShapeDtypeStruct((n_dst, 32), jnp.float32),
            jax.ShapeDtypeStruct((n_dst, 32), jnp.float32),
        ],
        mesh=_MESH,
        compiler_params=pltpu.CompilerParams(use_tc_tiling_on_sc=False),
        scratch_types=[
            pltpu.VMEM((SPC, CH), jnp.int32),
            pltpu.VMEM((SPC, CH), jnp.int32),
            pltpu.VMEM((HCHUNK, 32), jnp.float32),
            pltpu.VMEM_SHARED((n_dst, 32), jnp.float32),
            pltpu.SemaphoreType.DMA,
        ],
    )
    return fn(e3, y0, y1, zeros32)


BLK = 400


def _tc_mm1(x_flow, pe, te, wcat, bias, n_rows, n_out):
    grid = (n_rows // BLK,)

    def body(x_ref, pe_ref, te_ref, w_ref, b_ref, *outs):
        x = x_ref[...]
        w = w_ref[...]
        o = jnp.dot(x, w[:125, :], preferred_element_type=jnp.float32)
        o = o + pe_ref[...][:, :1] * w[125:126, :]
        o = o + jnp.dot(te_ref[...][:, :2], w[126:128, :],
                        preferred_element_type=jnp.float32)
        if n_out == 128:
            y = o[:, :64]
            outs[0][...] = y[:, :32]
            outs[1][...] = y[:, 32:]
            outs[2][...] = o[:, 64:] + b_ref[...]
        else:
            outs[0][...] = o[:, :32]
            outs[1][...] = o[:, 32:]

    if n_out == 128:
        out_shape = [
            jax.ShapeDtypeStruct((n_rows, 32), jnp.float32),
            jax.ShapeDtypeStruct((n_rows, 32), jnp.float32),
            jax.ShapeDtypeStruct((n_rows, 64), jnp.float32),
        ]
        out_specs = [
            pl.BlockSpec((BLK, 32), lambda i: (i, 0)),
            pl.BlockSpec((BLK, 32), lambda i: (i, 0)),
            pl.BlockSpec((BLK, 64), lambda i: (i, 0)),
        ]
    else:
        out_shape = [
            jax.ShapeDtypeStruct((n_rows, 32), jnp.float32),
            jax.ShapeDtypeStruct((n_rows, 32), jnp.float32),
        ]
        out_specs = [
            pl.BlockSpec((BLK, 32), lambda i: (i, 0)),
            pl.BlockSpec((BLK, 32), lambda i: (i, 0)),
        ]

    return pl.pallas_call(
        body,
        grid=grid,
        in_specs=[
            pl.BlockSpec((BLK, 125), lambda i: (i, 0)),
            pl.BlockSpec((BLK, 8), lambda i: (i, 0)),
            pl.BlockSpec((BLK, 8), lambda i: (i, 0)),
            pl.BlockSpec((128, n_out), lambda i: (0, 0)),
            pl.BlockSpec((1, 64), lambda i: (0, 0)),
        ],
        out_specs=out_specs,
        out_shape=out_shape,
    )(x_flow, pe, te, wcat, bias)


def _tc_combine_mm2(s0, s1, cnt, base1, w2, b2):
    grid = (NF // BLK,)

    def body(s0_ref, s1_ref, c_ref, base_ref, w_ref, b_ref, o0, o1, o2):
        ssum = jnp.concatenate([s0_ref[...], s1_ref[...]], axis=1)
        cnt = jnp.maximum(c_ref[...][:, :1], 1.0)
        f1 = jax.nn.relu(ssum / cnt + base_ref[...])
        o = jnp.dot(f1, w_ref[...], preferred_element_type=jnp.float32)
        o0[...] = o[:, :32]
        o1[...] = o[:, 32:64]
        o2[...] = o[:, 64:] + b_ref[...]

    return pl.pallas_call(
        body,
        grid=grid,
        in_specs=[
            pl.BlockSpec((BLK, 32), lambda i: (i, 0)),
            pl.BlockSpec((BLK, 32), lambda i: (i, 0)),
            pl.BlockSpec((BLK, 16), lambda i: (i, 0)),
            pl.BlockSpec((BLK, 64), lambda i: (i, 0)),
            pl.BlockSpec((64, 128), lambda i: (0, 0)),
            pl.BlockSpec((1, 64), lambda i: (0, 0)),
        ],
        out_specs=[
            pl.BlockSpec((BLK, 32), lambda i: (i, 0)),
            pl.BlockSpec((BLK, 32), lambda i: (i, 0)),
            pl.BlockSpec((BLK, 64), lambda i: (i, 0)),
        ],
        out_shape=[
            jax.ShapeDtypeStruct((NF, 32), jnp.float32),
            jax.ShapeDtypeStruct((NF, 32), jnp.float32),
            jax.ShapeDtypeStruct((NF, 64), jnp.float32),
        ],
    )(s0, s1, cnt, base1, w2, b2)


def _tc_host_mm2(s0, s1, cnt, b1fh, wh2):
    grid = (NH // BLK,)

    def body(s0_ref, s1_ref, c_ref, b_ref, w_ref, o0, o1):
        ssum = jnp.concatenate([s0_ref[...], s1_ref[...]], axis=1)
        cnt = jnp.maximum(c_ref[...][:, :1], 1.0)
        h1 = jax.nn.relu(ssum / cnt + b_ref[...])
        o = jnp.dot(h1, w_ref[...], preferred_element_type=jnp.float32)
        o0[...] = o[:, :32]
        o1[...] = o[:, 32:]

    return pl.pallas_call(
        body,
        grid=grid,
        in_specs=[
            pl.BlockSpec((BLK, 32), lambda i: (i, 0)),
            pl.BlockSpec((BLK, 32), lambda i: (i, 0)),
            pl.BlockSpec((BLK, 16), lambda i: (i, 0)),
            pl.BlockSpec((1, 64), lambda i: (0, 0)),
            pl.BlockSpec((64, 64), lambda i: (0, 0)),
        ],
        out_specs=[
            pl.BlockSpec((BLK, 32), lambda i: (i, 0)),
            pl.BlockSpec((BLK, 32), lambda i: (i, 0)),
        ],
        out_shape=[
            jax.ShapeDtypeStruct((NH, 32), jnp.float32),
            jax.ShapeDtypeStruct((NH, 32), jnp.float32),
        ],
    )(s0, s1, cnt, b1fh, wh2)


def _tc_final(s20, s21, cnt_ff, base2, sh0, sh1, cnt_hf, batch3,
              wc1, bc1, wc2, bc2, wc3, bc3):
    grid = (NF // BLK,)
    nhb = NH // BLK

    def hmap(i):
        return (jnp.minimum(i, nhb - 1), 0)

    def body(s20_ref, s21_ref, cf_ref, base_ref, sh0_ref, sh1_ref, ch_ref,
             b_ref, w1_ref, b1_ref, w2_ref, b2_ref, w3_ref, b3_ref,
             out_ref, acc):
        b = pl.program_id(0)

        @pl.when(b == 0)
        def _():
            acc[...] = jnp.full((NG, H), -jnp.inf, jnp.float32)

        ssum = jnp.concatenate([s20_ref[...], s21_ref[...]], axis=1)
        cf = jnp.maximum(cf_ref[...][:, :1], 1.0)
        f2 = ssum / cf + base_ref[...]

        sh = jnp.concatenate([sh0_ref[...], sh1_ref[...]], axis=1)
        chc = jnp.maximum(ch_ref[...][:, :1], 1.0)
        f2 = f2 + jnp.where(b < nhb, sh / chc, 0.0)

        seg = b_ref[0, 0, :].reshape(BLK, 1)
        lo = b_ref[0, 0, 0]
        hi = b_ref[0, 0, BLK - 1]

        def upd(g, _):
            m = seg == g
            colmax = jnp.max(jnp.where(m, f2, -jnp.inf), axis=0,
                             keepdims=True)
            acc[pl.ds(g, 1), :] = jnp.maximum(acc[pl.ds(g, 1), :], colmax)
            return 0

        lax.fori_loop(lo, hi + 1, upd, 0)

        @pl.when(b == grid[0] - 1)
        def _():
            pooled = acc[...]
            o = jax.nn.relu(jnp.dot(pooled, w1_ref[...],
                                    preferred_element_type=jnp.float32)
                            + b1_ref[...])
            o = jax.nn.relu(jnp.dot(o, w2_ref[...],
                                    preferred_element_type=jnp.float32)
                            + b2_ref[...])
            out_ref[...] = jnp.dot(o, w3_ref[...],
                                   preferred_element_type=jnp.float32) \
                + b3_ref[...]

    return pl.pallas_call(
        body,
        grid=grid,
        in_specs=[
            pl.BlockSpec((BLK, 32), lambda i: (i, 0)),
            pl.BlockSpec((BLK, 32), lambda i: (i, 0)),
            pl.BlockSpec((BLK, 16), lambda i: (i, 0)),
            pl.BlockSpec((BLK, 64), lambda i: (i, 0)),
            pl.BlockSpec((BLK, 32), hmap),
            pl.BlockSpec((BLK, 32), hmap),
            pl.BlockSpec((BLK, 16), hmap),
            pl.BlockSpec((1, 1, BLK), lambda i: (i, 0, 0)),
            pl.BlockSpec((64, 64), lambda i: (0, 0)),
            pl.BlockSpec((1, 64), lambda i: (0, 0)),
            pl.BlockSpec((64, 64), lambda i: (0, 0)),
            pl.BlockSpec((1, 64), lambda i: (0, 0)),
            pl.BlockSpec((64, 16), lambda i: (0, 0)),
            pl.BlockSpec((1, 16), lambda i: (0, 0)),
        ],
        out_specs=pl.BlockSpec((NG, 16), lambda i: (0, 0)),
        out_shape=jax.ShapeDtypeStruct((NG, 16), jnp.float32),
        scratch_shapes=[pltpu.VMEM((NG, H), jnp.float32)],
        compiler_params=pltpu.CompilerParams(
            dimension_semantics=("arbitrary",)),
    )(s20, s21, cnt_ff, base2, sh0, sh1, cnt_hf, batch3,
      wc1, bc1, wc2, bc2, wc3, bc3)


def kernel(x_flow, port_table, tcp_table, Wl1_hf, Wr1_hf, b1_hf, Wl1_fh,
           Wr1_fh, b1_fh, Wl1_ff, Wr1_ff, b1_ff, Wl2_hf, Wr2_hf, b2_hf,
           Wl2_ff, Wr2_ff, b2_ff, Wc1, bc1, Wc2, bc2, Wc3, bc3, dst_ports,
           tcp_flags, edge_index_hf, edge_index_fh, edge_index_ff, batch,
           num_hosts):
    f32 = jnp.float32
    i32 = jnp.int32

    wcat1 = jnp.concatenate([Wl1_ff.T, (Wr1_hf + Wr1_ff).T], axis=1)
    b1 = (b1_hf + b1_ff).reshape(1, 64)
    wz1 = Wl1_fh.T
    w2cat = jnp.concatenate([Wl2_ff.T, (Wr2_hf + Wr2_ff).T], axis=1)
    b2 = (b2_hf + b2_ff).reshape(1, 64)
    wh2 = Wl2_hf.T
    b1fh = b1_fh.reshape(1, 64)
    wc1 = Wc1.T
    wc2 = Wc2.T
    wc3 = jnp.zeros((64, 16), f32).at[:, :10].set(Wc3.T)
    bc3p = jnp.zeros((1, 16), f32).at[:, :10].set(bc3.reshape(1, 10))
    bc1r = bc1.reshape(1, 64)
    bc2r = bc2.reshape(1, 64)

    ports2 = dst_ports.astype(i32).reshape(NF // CH, CH)
    tcps2 = tcp_flags.astype(i32).reshape(NF // CH, CH)
    eff3 = edge_index_ff.astype(i32).reshape(-1, CH)
    efh3 = edge_index_fh.astype(i32).reshape(-1, CH)
    ehf3 = edge_index_hf.astype(i32).reshape(-1, CH)
    batch3 = batch.astype(i32).reshape(NF // BLK, 1, BLK)

    ones16 = jnp.ones((CH, 16), f32)
    zeros16 = jnp.zeros((CHUNK, 16), f32)
    zeros32 = jnp.zeros((HCHUNK, 32), f32)

    ptab8 = jnp.pad(port_table, ((0, 0), (0, 7)))
    ttab8 = jnp.pad(tcp_table, ((0, 0), (0, 6)))
    pe, te, cnt_ff, cnt_fh, cnt_hf = _sc_embed_counts(
        ports2, tcps2, eff3, efh3, ehf3, ptab8, ttab8, ones16, zeros16)

    y10, y11, base1 = _tc_mm1(x_flow, pe, te, wcat1, b1, NF, 128)
    z10, z11 = _tc_mm1(x_flow, pe, te, wz1, b1, NH, 64)

    sff10, sff11 = _sc_segsum(eff3, y10, y11, zeros32, NF, 12 * NF)
    sfh0, sfh1 = _sc_segsum(efh3, z10, z11, zeros32, NH, 2 * NH * 5)

    y20, y21, base2 = _tc_combine_mm2(sff10, sff11, cnt_ff, base1, w2cat, b2)
    w20, w21 = _tc_host_mm2(sfh0, sfh1, cnt_fh, b1fh, wh2)

    sff20, sff21 = _sc_segsum(eff3, y20, y21, zeros32, NF, 12 * NF)
    shf0, shf1 = _sc_segsum(ehf3, w20, w21, zeros32, NH, 2 * NH * 5)

    out16 = _tc_final(sff20, sff21, cnt_ff, base2, shf0, shf1, cnt_hf,
                      batch3, wc1, bc1r, wc2, bc2r, wc3, bc3p)
    return out16[:, :10]
